# baseline (speedup 1.0000x reference)
import functools

import jax
import jax.numpy as jnp
from jax import lax
from jax.experimental import pallas as pl
from jax.experimental.pallas import tpu as pltpu
from jax.experimental.pallas import tpu_sc as plsc

_NC = 2
_NS = 16
_NW = _NC * _NS
_L = 16
_NBUF = 3


def _transe_sc(h_head, h_tail, rels, rel_emb):
    B, F = h_head.shape
    bpw = B // _NW
    C = min(bpw, 128)
    nchunks = bpw // C
    nbuf = min(_NBUF, nchunks)
    mesh = plsc.VectorSubcoreMesh(core_axis_name="c", subcore_axis_name="s")

    scratch = dict(
        idx_v=pltpu.VMEM((bpw,), jnp.int32),
        out_v=pltpu.VMEM((bpw,), jnp.float32),
        hsem=pltpu.SemaphoreType.DMA((nbuf,)),
        tsem=pltpu.SemaphoreType.DMA((nbuf,)),
        rsem=pltpu.SemaphoreType.DMA((nbuf,)),
    )
    for i in range(nbuf):
        scratch[f"hp{i}"] = pltpu.VMEM((C, F), jnp.float32)
        scratch[f"t{i}"] = pltpu.VMEM((C, F), jnp.float32)

    @functools.partial(
        pl.kernel,
        out_type=jax.ShapeDtypeStruct((B,), jnp.float32),
        mesh=mesh,
        scratch_types=scratch,
        compiler_params=pltpu.CompilerParams(needs_layout_passes=False),
    )
    def k(head_hbm, tail_hbm, rels_hbm, emb_hbm, out_hbm,
          idx_v, out_v, hsem, tsem, rsem, **bufs):
        hp = [bufs[f"hp{i}"] for i in range(nbuf)]
        tb = [bufs[f"t{i}"] for i in range(nbuf)]
        wid = lax.axis_index("s") * _NC + lax.axis_index("c")
        base = wid * bpw
        pltpu.sync_copy(rels_hbm.at[pl.ds(base, bpw)], idx_v)
        lanes = lax.iota(jnp.int32, _L)

        def issue_ht(g):
            s = g % nbuf
            ch = pltpu.async_copy(head_hbm.at[pl.ds(base + g * C, C)],
                                  hp[s], hsem.at[s])
            ct = pltpu.async_copy(tail_hbm.at[pl.ds(base + g * C, C)],
                                  tb[s], tsem.at[s])
            return ch, ct

        def issue_r(g):
            s = g % nbuf
            return pltpu.async_copy(emb_hbm.at[idx_v.at[pl.ds(g * C, C)]],
                                    hp[s], rsem.at[s], add=True)

        cph, cpt, cpr = {}, {}, {}
        cph[0], cpt[0] = issue_ht(0)
        cph[0].wait()
        cpr[0] = issue_r(0)
        if nchunks > 1:
            cph[1], cpt[1] = issue_ht(1)

        for g in range(nchunks):
            if g + 1 < nchunks:
                cph[g + 1].wait()
                cpr[g + 1] = issue_r(g + 1)
            if g + 2 < nchunks:
                cph[g + 2], cpt[g + 2] = issue_ht(g + 2)
            cpr[g].wait()
            cpt[g].wait()
            s = g % nbuf
            hp_g, t_g = hp[s], tb[s]
            for g2 in range(C // _L):
                rowi = g2 * _L + lanes

                def body(f, acc, hp_g=hp_g, t_g=t_g, rowi=rowi):
                    colf = jnp.full((_L,), f, jnp.int32)
                    hv = plsc.load_gather(hp_g, [rowi, colf])
                    tv = plsc.load_gather(t_g, [rowi, colf])
                    return acc + jnp.abs(hv - tv)

                acc = lax.fori_loop(0, F, body, jnp.zeros((_L,), jnp.float32),
                                    unroll=8)
                out_v[pl.ds(g * C + g2 * _L, _L)] = -acc
        pltpu.sync_copy(out_v, out_hbm.at[pl.ds(base, bpw)])

    return k(h_head, h_tail, rels, rel_emb)


def kernel(h_head, h_tail, rels, rel_emb):
    return _transe_sc(h_head, h_tail, rels.astype(jnp.int32), rel_emb)


# speedup vs baseline: 2.2612x; 2.2612x over previous
import functools

import jax
import jax.numpy as jnp
from jax import lax
from jax.experimental import pallas as pl
from jax.experimental.pallas import tpu as pltpu
from jax.experimental.pallas import tpu_sc as plsc

_NC = 2
_NS = 16
_NW = _NC * _NS
_L = 16
_NBUF = 3


def _transe_sc(h_head, h_tail, rels, rel_emb):
    B, F = h_head.shape
    bpw = B // _NW
    C = min(bpw, 128)
    nchunks = bpw // C
    nbuf = min(_NBUF, nchunks)
    mesh = plsc.VectorSubcoreMesh(core_axis_name="c", subcore_axis_name="s")

    scratch = dict(
        idx_v=pltpu.VMEM((bpw,), jnp.int32),
        out_v=pltpu.VMEM((bpw,), jnp.float32),
        tr_v=pltpu.VMEM((_L * (_L + 1),), jnp.float32),
        hsem=pltpu.SemaphoreType.DMA((nbuf,)),
        tsem=pltpu.SemaphoreType.DMA((nbuf,)),
        rsem=pltpu.SemaphoreType.DMA((nbuf,)),
    )
    for i in range(nbuf):
        scratch[f"hp{i}"] = pltpu.VMEM((C, F), jnp.float32)
        scratch[f"t{i}"] = pltpu.VMEM((C, F), jnp.float32)

    @functools.partial(
        pl.kernel,
        out_type=jax.ShapeDtypeStruct((B,), jnp.float32),
        mesh=mesh,
        scratch_types=scratch,
        compiler_params=pltpu.CompilerParams(needs_layout_passes=False),
    )
    def k(head_hbm, tail_hbm, rels_hbm, emb_hbm, out_hbm,
          idx_v, out_v, tr_v, hsem, tsem, rsem, **bufs):
        hp = [bufs[f"hp{i}"] for i in range(nbuf)]
        tb = [bufs[f"t{i}"] for i in range(nbuf)]
        wid = lax.axis_index("s") * _NC + lax.axis_index("c")
        base = wid * bpw
        pltpu.sync_copy(rels_hbm.at[pl.ds(base, bpw)], idx_v)
        lanes = lax.iota(jnp.int32, _L)

        def issue_ht(g):
            s = g % nbuf
            ch = pltpu.async_copy(head_hbm.at[pl.ds(base + g * C, C)],
                                  hp[s], hsem.at[s])
            ct = pltpu.async_copy(tail_hbm.at[pl.ds(base + g * C, C)],
                                  tb[s], tsem.at[s])
            return ch, ct

        def issue_r(g):
            s = g % nbuf
            return pltpu.async_copy(emb_hbm.at[idx_v.at[pl.ds(g * C, C)]],
                                    hp[s], rsem.at[s], add=True)

        cph, cpt, cpr = {}, {}, {}
        cph[0], cpt[0] = issue_ht(0)
        cph[0].wait()
        cpr[0] = issue_r(0)
        if nchunks > 1:
            cph[1], cpt[1] = issue_ht(1)

        for g in range(nchunks):
            if g + 1 < nchunks:
                cph[g + 1].wait()
                cpr[g + 1] = issue_r(g + 1)
            if g + 2 < nchunks:
                cph[g + 2], cpt[g + 2] = issue_ht(g + 2)
            cpr[g].wait()
            cpt[g].wait()
            s = g % nbuf
            hp_g, t_g = hp[s], tb[s]
            for g2 in range(C // _L):

                def row_body(r2, carry, hp_g=hp_g, t_g=t_g, g2=g2):
                    row = g2 * _L + r2
                    acc = jnp.zeros((_L,), jnp.float32)
                    for v in range(F // _L):
                        hv = hp_g[row, pl.ds(v * _L, _L)]
                        tv = t_g[row, pl.ds(v * _L, _L)]
                        acc = acc + jnp.abs(hv - tv)
                    plsc.store_scatter(tr_v, [lanes * (_L + 1) + r2], acc)
                    return carry

                lax.fori_loop(0, _L, row_body, 0)
                out_acc = tr_v[pl.ds(0, _L)]
                for j in range(1, _L):
                    out_acc = out_acc + tr_v[pl.ds(j * (_L + 1), _L)]
                out_v[pl.ds(g * C + g2 * _L, _L)] = -out_acc
        pltpu.sync_copy(out_v, out_hbm.at[pl.ds(base, bpw)])

    return k(h_head, h_tail, rels, rel_emb)


def kernel(h_head, h_tail, rels, rel_emb):
    return _transe_sc(h_head, h_tail, rels.astype(jnp.int32), rel_emb)


# speedup vs baseline: 2.7399x; 1.2117x over previous
import functools

import jax
import jax.numpy as jnp
from jax import lax
from jax.experimental import pallas as pl
from jax.experimental.pallas import tpu as pltpu
from jax.experimental.pallas import tpu_sc as plsc

_NC = 2
_NS = 16
_NW = _NC * _NS
_L = 16
_NBUF = 3


def _transe_sc(h_head, h_tail, rels, rel_emb):
    B, F = h_head.shape
    bpw = B // _NW
    C = min(bpw, 128)
    nchunks = bpw // C
    nbuf = min(_NBUF, nchunks)
    W = C + 1
    mesh = plsc.VectorSubcoreMesh(core_axis_name="c", subcore_axis_name="s")

    scratch = dict(
        idx_v=pltpu.VMEM((bpw,), jnp.int32),
        out_v=pltpu.VMEM((bpw,), jnp.float32),
        tr_v=pltpu.VMEM((_L * W,), jnp.float32),
        isem=pltpu.SemaphoreType.DMA,
        hsem=pltpu.SemaphoreType.DMA((nbuf,)),
        tsem=pltpu.SemaphoreType.DMA((nbuf,)),
        rsem=pltpu.SemaphoreType.DMA((nbuf,)),
    )
    for i in range(nbuf):
        scratch[f"hp{i}"] = pltpu.VMEM((C, F), jnp.float32)
        scratch[f"t{i}"] = pltpu.VMEM((C, F), jnp.float32)

    @functools.partial(
        pl.kernel,
        out_type=jax.ShapeDtypeStruct((B,), jnp.float32),
        mesh=mesh,
        scratch_types=scratch,
        compiler_params=pltpu.CompilerParams(needs_layout_passes=False),
    )
    def k(head_hbm, tail_hbm, rels_hbm, emb_hbm, out_hbm,
          idx_v, out_v, tr_v, isem, hsem, tsem, rsem, **bufs):
        hp = [bufs[f"hp{i}"] for i in range(nbuf)]
        tb = [bufs[f"t{i}"] for i in range(nbuf)]
        wid = lax.axis_index("s") * _NC + lax.axis_index("c")
        base = wid * bpw
        lanes = lax.iota(jnp.int32, _L)
        lanes_w = lanes * W

        def issue_ht(g):
            s = g % nbuf
            ch = pltpu.async_copy(head_hbm.at[pl.ds(base + g * C, C)],
                                  hp[s], hsem.at[s])
            ct = pltpu.async_copy(tail_hbm.at[pl.ds(base + g * C, C)],
                                  tb[s], tsem.at[s])
            return ch, ct

        def issue_r(g):
            s = g % nbuf
            return pltpu.async_copy(emb_hbm.at[idx_v.at[pl.ds(g * C, C)]],
                                    hp[s], rsem.at[s], add=True)

        cph, cpt, cpr = {}, {}, {}
        cph[0], cpt[0] = issue_ht(0)
        cpi = pltpu.async_copy(rels_hbm.at[pl.ds(base, bpw)], idx_v, isem)
        cpi.wait()
        cph[0].wait()
        cpr[0] = issue_r(0)
        if nchunks > 1:
            cph[1], cpt[1] = issue_ht(1)

        for g in range(nchunks):
            if g + 1 < nchunks:
                cph[g + 1].wait()
                cpr[g + 1] = issue_r(g + 1)
            if g + 2 < nchunks:
                cph[g + 2], cpt[g + 2] = issue_ht(g + 2)
            cpr[g].wait()
            cpt[g].wait()
            s = g % nbuf
            hp_g, t_g = hp[s], tb[s]

            @plsc.parallel_loop(0, C, unroll=2)
            def _(r, hp_g=hp_g, t_g=t_g):
                d = [jnp.abs(hp_g[r, pl.ds(v * _L, _L)] -
                             t_g[r, pl.ds(v * _L, _L)])
                     for v in range(F // _L)]
                while len(d) > 1:
                    d = [a + b for a, b in zip(d[::2], d[1::2])]
                plsc.store_scatter(tr_v, [lanes_w + r], d[0])

            def group_body(g2, carry, g=g):
                col0 = g2 * _L
                out_acc = tr_v[pl.ds(col0, _L)]
                for j in range(1, _L):
                    out_acc = out_acc + tr_v[pl.ds(j * W + col0, _L)]
                out_v[pl.ds(g * C + g2 * _L, _L)] = -out_acc
                return carry

            lax.fori_loop(0, C // _L, group_body, 0)
        pltpu.sync_copy(out_v, out_hbm.at[pl.ds(base, bpw)])

    return k(h_head, h_tail, rels, rel_emb)


def kernel(h_head, h_tail, rels, rel_emb):
    return _transe_sc(h_head, h_tail, rels.astype(jnp.int32), rel_emb)


# speedup vs baseline: 2.8171x; 1.0282x over previous
import functools

import jax
import jax.numpy as jnp
from jax import lax
from jax.experimental import pallas as pl
from jax.experimental.pallas import tpu as pltpu
from jax.experimental.pallas import tpu_sc as plsc

_NC = 2
_NS = 16
_NW = _NC * _NS
_L = 16
_NBUF = 3


def _transe_sc(h_head, h_tail, rels, rel_emb):
    B, F = h_head.shape
    bpw = B // _NW
    C = min(bpw, 128)
    nchunks = bpw // C
    nbuf = min(_NBUF, nchunks)
    W = C + 1
    mesh = plsc.VectorSubcoreMesh(core_axis_name="c", subcore_axis_name="s")

    N = rel_emb.shape[0]
    scratch = dict(
        idx_v=pltpu.VMEM((bpw,), jnp.int32),
        out_v=pltpu.VMEM((bpw,), jnp.float32),
        tr_v=pltpu.VMEM((_L * W,), jnp.float32),
        tbl=pltpu.VMEM_SHARED((N, F), jnp.float32),
        isem=pltpu.SemaphoreType.DMA,
        hsem=pltpu.SemaphoreType.DMA((nbuf,)),
        tsem=pltpu.SemaphoreType.DMA((nbuf,)),
        rsem=pltpu.SemaphoreType.DMA((nbuf,)),
    )
    for i in range(nbuf):
        scratch[f"hp{i}"] = pltpu.VMEM((C, F), jnp.float32)
        scratch[f"t{i}"] = pltpu.VMEM((C, F), jnp.float32)

    @functools.partial(
        pl.kernel,
        out_type=jax.ShapeDtypeStruct((B,), jnp.float32),
        mesh=mesh,
        scratch_types=scratch,
        compiler_params=pltpu.CompilerParams(needs_layout_passes=False),
    )
    def k(head_hbm, tail_hbm, rels_hbm, emb_hbm, out_hbm,
          idx_v, out_v, tr_v, tbl, isem, hsem, tsem, rsem, **bufs):
        hp = [bufs[f"hp{i}"] for i in range(nbuf)]
        tb = [bufs[f"t{i}"] for i in range(nbuf)]
        wid = lax.axis_index("s") * _NC + lax.axis_index("c")
        base = wid * bpw
        lanes = lax.iota(jnp.int32, _L)
        lanes_w = lanes * W

        def issue_ht(g):
            s = g % nbuf
            ch = pltpu.async_copy(head_hbm.at[pl.ds(base + g * C, C)],
                                  hp[s], hsem.at[s])
            ct = pltpu.async_copy(tail_hbm.at[pl.ds(base + g * C, C)],
                                  tb[s], tsem.at[s])
            return ch, ct

        def issue_r(g):
            s = g % nbuf
            return pltpu.async_copy(tbl.at[idx_v.at[pl.ds(g * C, C)]],
                                    hp[s], rsem.at[s], add=True)

        cph, cpt, cpr = {}, {}, {}
        cph[0], cpt[0] = issue_ht(0)
        cpi = pltpu.async_copy(rels_hbm.at[pl.ds(base, bpw)], idx_v, isem)

        @pl.when(lax.axis_index("s") == 0)
        def _():
            pltpu.sync_copy(emb_hbm, tbl)

        plsc.subcore_barrier()
        cpi.wait()
        cph[0].wait()
        cpr[0] = issue_r(0)
        if nchunks > 1:
            cph[1], cpt[1] = issue_ht(1)

        for g in range(nchunks):
            if g + 1 < nchunks:
                cph[g + 1].wait()
                cpr[g + 1] = issue_r(g + 1)
            if g + 2 < nchunks:
                cph[g + 2], cpt[g + 2] = issue_ht(g + 2)
            cpr[g].wait()
            cpt[g].wait()
            s = g % nbuf
            hp_g, t_g = hp[s], tb[s]

            @plsc.parallel_loop(0, C, unroll=4)
            def _(r, hp_g=hp_g, t_g=t_g):
                d = [jnp.abs(hp_g[r, pl.ds(v * _L, _L)] -
                             t_g[r, pl.ds(v * _L, _L)])
                     for v in range(F // _L)]
                while len(d) > 1:
                    d = [a + b for a, b in zip(d[::2], d[1::2])]
                plsc.store_scatter(tr_v, [lanes_w + r], d[0])

            def group_body(g2, carry, g=g):
                col0 = g2 * _L
                out_acc = tr_v[pl.ds(col0, _L)]
                for j in range(1, _L):
                    out_acc = out_acc + tr_v[pl.ds(j * W + col0, _L)]
                out_v[pl.ds(g * C + g2 * _L, _L)] = -out_acc
                return carry

            lax.fori_loop(0, C // _L, group_body, 0)
        pltpu.sync_copy(out_v, out_hbm.at[pl.ds(base, bpw)])

    return k(h_head, h_tail, rels, rel_emb)


def kernel(h_head, h_tail, rels, rel_emb):
    return _transe_sc(h_head, h_tail, rels.astype(jnp.int32), rel_emb)
